# baseline (speedup 1.0000x reference)
import functools
import math

import jax
import jax.numpy as jnp
from jax import lax
from jax.experimental import pallas as pl
from jax.experimental.pallas import tpu as pltpu
from jax.experimental.pallas import tpu_sc as plsc

_S = 64.0
_M2 = 0.5
_COS_M = math.cos(_M2)
_SIN_M = math.sin(_M2)
_THETA = math.cos(math.pi - _M2)
_SINMM = math.sin(math.pi - _M2) * _M2

_B = 1024
_V = 100000

_NC = 2
_NS = 16
_L = 16
_NW = _NC * _NS
_RPW = _B // _NW

_BC = 2048


def _sc_gather_body(flat_hbm, labels_hbm, out_hbm, lab_v, idx_v, val_v, sem):
    wid = lax.axis_index("s") * _NC + lax.axis_index("c")
    base = wid * _RPW
    pltpu.sync_copy(labels_hbm.at[pl.ds(base, _RPW)], lab_v)
    for j in range(_RPW // _L):
        lab = jnp.maximum(lab_v[pl.ds(j * _L, _L)], 0)
        rows = (base + j * _L) + lax.iota(jnp.int32, _L)
        idx_v[pl.ds(j * _L, _L)] = rows * _V + lab
    pltpu.async_copy(flat_hbm.at[idx_v], val_v, sem).wait()
    pltpu.sync_copy(val_v, out_hbm.at[pl.ds(base, _RPW)])


@functools.cache
def _sc_gather():
    return functools.partial(
        pl.kernel,
        mesh=plsc.VectorSubcoreMesh(core_axis_name="c", subcore_axis_name="s"),
        out_type=jax.ShapeDtypeStruct((_B,), jnp.float32),
        scratch_types=[
            pltpu.VMEM((_RPW,), jnp.int32),
            pltpu.VMEM((_RPW,), jnp.int32),
            pltpu.VMEM((_RPW,), jnp.float32),
            pltpu.SemaphoreType.DMA,
        ],
    )(_sc_gather_body)


def _merge_body(lab_ref, t_ref, x_ref, o_ref):
    j = pl.program_id(0)
    x = x_ref[...]
    lab = lab_ref[...]
    t = t_ref[...]
    sin_t = jnp.sqrt(1.0 - t * t)
    cos_theta_m = t * _COS_M - sin_t * _SIN_M
    f = jnp.where(t > _THETA, cos_theta_m, t - _SINMM)
    upd = jnp.where(lab >= 0, f, t)
    cols = lax.broadcasted_iota(jnp.int32, x.shape, 1) + j * _BC
    o_ref[...] = _S * jnp.where(cols == lab, upd, x)


def _tc_merge(logits, lab2, t2):
    b, v = logits.shape
    return pl.pallas_call(
        _merge_body,
        grid=(pl.cdiv(v, _BC),),
        in_specs=[
            pl.BlockSpec((b, 1), lambda j: (0, 0)),
            pl.BlockSpec((b, 1), lambda j: (0, 0)),
            pl.BlockSpec((b, _BC), lambda j: (0, j)),
        ],
        out_specs=pl.BlockSpec((b, _BC), lambda j: (0, j)),
        out_shape=jax.ShapeDtypeStruct((b, v), jnp.float32),
    )(lab2, t2, logits)


def kernel(logits, labels):
    b, v = logits.shape
    t = _sc_gather()(logits.reshape(-1), labels)
    return _tc_merge(logits, labels.reshape(b, 1), t.reshape(b, 1))


# speedup vs baseline: 1.6047x; 1.6047x over previous
import functools
import math

import jax
import jax.numpy as jnp
from jax import lax
from jax.experimental import pallas as pl
from jax.experimental.pallas import tpu as pltpu
from jax.experimental.pallas import tpu_sc as plsc

_S = 64.0
_M2 = 0.5
_COS_M = math.cos(_M2)
_SIN_M = math.sin(_M2)
_THETA = math.cos(math.pi - _M2)
_SINMM = math.sin(math.pi - _M2) * _M2

_BC = 2048


def _merge_body(lab_ref, x_ref, o_ref):
    j = pl.program_id(0)
    x = x_ref[...]
    lab = lab_ref[...]
    cols = lax.broadcasted_iota(jnp.int32, x.shape, 1) + j * _BC
    mask = cols == lab
    t = jnp.sum(jnp.where(mask, x, 0.0), axis=1, keepdims=True)
    sin_t = jnp.sqrt(1.0 - t * t)
    cos_theta_m = t * _COS_M - sin_t * _SIN_M
    f = jnp.where(t > _THETA, cos_theta_m, t - _SINMM)
    upd = jnp.where(lab >= 0, f, t)
    o_ref[...] = _S * jnp.where(mask, upd, x)


def kernel(logits, labels):
    b, v = logits.shape
    return pl.pallas_call(
        _merge_body,
        grid=(pl.cdiv(v, _BC),),
        in_specs=[
            pl.BlockSpec((b, 1), lambda j: (0, 0)),
            pl.BlockSpec((b, _BC), lambda j: (0, j)),
        ],
        out_specs=pl.BlockSpec((b, _BC), lambda j: (0, j)),
        out_shape=jax.ShapeDtypeStruct((b, v), jnp.float32),
    )(labels.reshape(b, 1), logits)
